# baseline (speedup 1.0000x reference)
import functools

import jax
import jax.numpy as jnp
from jax import lax
from jax.experimental import pallas as pl
from jax.experimental.pallas import tpu as pltpu
from jax.experimental.pallas import tpu_sc as plsc

N_FIELDS = 26
VOCAB = 100000
EMB = 32
BATCH = 4096
HIST = 20
WIDE_DIM = N_FIELDS * VOCAB

NC = 2
NS = 16
L = 16
NW = NC * NS

VPAD = 782 * 128
BC = 128
NCHUNK = BATCH // BC
FIELD_GROUPS = (2, 4, 8, 12)

WROWS = 20320
WIDE_PER_W = BATCH * N_FIELDS // NW
WBURST = 32
NWBURST = WIDE_PER_W // WBURST

_mesh = plsc.VectorSubcoreMesh(core_axis_name="c", subcore_axis_name="s")


def _deep_body(nf, xd_hbm, deep_hbm, out_hbm,
               vline_v, idx0_v, idx1_v, outc_v, sem_i, sem_o, wid):
    idx_slots = (idx0_v, idx1_v)

    @pl.loop(0, nf)
    def _task(t):
        gt = wid * nf + t
        f = gt >> 5
        e = gt & 31
        pltpu.sync_copy(deep_hbm.at[f, e], vline_v)
        pltpu.async_copy(xd_hbm.at[f, :, pl.ds(0, BC)], idx0_v, sem_i)
        @pl.when(t > 0)
        def _():
            pltpu.make_async_copy(outc_v, out_hbm.at[f, e], sem_o).wait()

        @pl.loop(0, NCHUNK // 2)
        def _chunkpair(p):
            for s in range(2):
                c = p * 2 + s
                pltpu.make_async_copy(
                    xd_hbm.at[f, :, pl.ds(0, BC)], idx_slots[s], sem_i,
                ).wait()

                @pl.when(c + 1 < NCHUNK)
                def _():
                    pltpu.async_copy(
                        xd_hbm.at[f, :, pl.ds((c + 1) * BC, BC)],
                        idx_slots[1 - s], sem_i)

                idx_v = idx_slots[s]
                for g in range(BC // L):
                    a0 = jnp.zeros((L,), jnp.float32)
                    a1 = jnp.zeros((L,), jnp.float32)
                    for h in range(HIST):
                        iv = idx_v[h, pl.ds(g * L, L)]
                        val = plsc.load_gather(vline_v, [iv >> 7, iv & 127])
                        if h & 1:
                            a1 = a1 + val
                        else:
                            a0 = a0 + val
                    outc_v[pl.ds(c * BC + g * L, L)] = a0 + a1

        pltpu.async_copy(outc_v, out_hbm.at[f, e], sem_o)

    pltpu.make_async_copy(outc_v, out_hbm.at[0, 0], sem_o).wait()


def _wide_body(xw_hbm, wide_hbm, wpart_hbm,
               widx_v, wrow_v, wv0_v, wv1_v, wacc_v, sem_w, wid):
    pltpu.sync_copy(xw_hbm.at[pl.ds(wid * (NWBURST // 4), NWBURST // 4)],
                    widx_v)

    @pl.loop(0, NWBURST // 4)
    def _rowids(r):
        for g in range(128 // L):
            wrow_v[r, pl.ds(g * L, L)] = widx_v[r, pl.ds(g * L, L)] >> 7

    lane = lax.iota(jnp.int32, L)
    wacc_v[...] = jnp.zeros((L,), jnp.float32)
    wv_slots = (wv0_v, wv1_v)
    pltpu.async_copy(wide_hbm.at[wrow_v.at[0, pl.ds(0, WBURST)]], wv0_v,
                     sem_w)

    @pl.loop(0, NWBURST // 2)
    def _wpair(p):
        for s in range(2):
            b = p * 2 + s
            r = b >> 2
            q = (b & 3) * WBURST
            pltpu.make_async_copy(
                wide_hbm.at[wrow_v.at[0, pl.ds(0, WBURST)]], wv_slots[s],
                sem_w).wait()

            @pl.when(b + 1 < NWBURST)
            def _():
                b1 = b + 1
                pltpu.async_copy(
                    wide_hbm.at[wrow_v.at[b1 >> 2,
                                          pl.ds((b1 & 3) * WBURST, WBURST)]],
                    wv_slots[1 - s], sem_w)

            acc = wacc_v[...]
            for g in range(WBURST // L):
                cols = widx_v[r, pl.ds(q + g * L, L)] & 127
                acc = acc + plsc.load_gather(wv_slots[s],
                                             [lane + g * L, cols])
            wacc_v[...] = acc

    pltpu.sync_copy(wacc_v, wpart_hbm.at[wid])


_DEEP_SCRATCH = [
    pltpu.VMEM((VPAD // 128, 128), jnp.float32),
    pltpu.VMEM((HIST, BC), jnp.int32),
    pltpu.VMEM((HIST, BC), jnp.int32),
    pltpu.VMEM((BATCH,), jnp.float32),
    pltpu.SemaphoreType.DMA,
    pltpu.SemaphoreType.DMA,
]
_WIDE_SCRATCH = [
    pltpu.VMEM((NWBURST // 4, 128), jnp.int32),
    pltpu.VMEM((NWBURST // 4, 128), jnp.int32),
    pltpu.VMEM((WBURST, 128), jnp.float32),
    pltpu.VMEM((WBURST, 128), jnp.float32),
    pltpu.VMEM((L,), jnp.float32),
    pltpu.SemaphoreType.DMA,
]
_CP = pltpu.CompilerParams(use_tc_tiling_on_sc=False,
                           needs_layout_passes=False)


@functools.cache
def _make_group_kernel(nf, with_wide):
    if with_wide:
        out_type = (jax.ShapeDtypeStruct((nf, EMB, BATCH), jnp.float32),
                    jax.ShapeDtypeStruct((NW, L), jnp.float32))
        scratch = _DEEP_SCRATCH + _WIDE_SCRATCH

        def body(xd_hbm, xw_hbm, deep_hbm, wide_hbm, out_hbm, wpart_hbm,
                 vline_v, idx0_v, idx1_v, outc_v, sem_i, sem_o,
                 widx_v, wrow_v, wv0_v, wv1_v, wacc_v, sem_w):
            wid = lax.axis_index("s") * NC + lax.axis_index("c")
            _deep_body(nf, xd_hbm, deep_hbm, out_hbm,
                       vline_v, idx0_v, idx1_v, outc_v, sem_i, sem_o, wid)
            _wide_body(xw_hbm, wide_hbm, wpart_hbm,
                       widx_v, wrow_v, wv0_v, wv1_v, wacc_v, sem_w, wid)
    else:
        out_type = jax.ShapeDtypeStruct((nf, EMB, BATCH), jnp.float32)
        scratch = _DEEP_SCRATCH

        def body(xd_hbm, deep_hbm, out_hbm,
                 vline_v, idx0_v, idx1_v, outc_v, sem_i, sem_o):
            wid = lax.axis_index("s") * NC + lax.axis_index("c")
            _deep_body(nf, xd_hbm, deep_hbm, out_hbm,
                       vline_v, idx0_v, idx1_v, outc_v, sem_i, sem_o, wid)

    return pl.kernel(body, out_type=out_type, mesh=_mesh,
                     compiler_params=_CP, scratch_types=scratch)


def _finish_body(p_ref, o_ref):
    o_ref[0, 0] = jax.nn.sigmoid(jnp.sum(p_ref[...]))


_finish = pl.pallas_call(
    _finish_body,
    out_specs=pl.BlockSpec(memory_space=pltpu.SMEM),
    out_shape=jax.ShapeDtypeStruct((1, 1), jnp.float32),
)


def kernel(X_w, X_d, deep_tables, wide_table):
    xdt = X_d.transpose(0, 2, 1)
    xw = X_w.reshape(BATCH * N_FIELDS // 128, 128)
    wt = jnp.pad(wide_table, ((0, WROWS * 128 - (WIDE_DIM + 1)), (0, 0)))
    wt = wt.reshape(WROWS, 128)

    dpt = deep_tables.transpose(0, 2, 1)
    pieces, wpart = [], None
    f0, prev = 0, None
    for i, nf in enumerate(FIELD_GROUPS):
        tbl = dpt
        if prev is not None:
            tbl = lax.optimization_barrier((tbl, prev))[0]
        dp4 = jnp.pad(tbl[f0:f0 + nf],
                      ((0, 0), (0, 0), (0, VPAD - (VOCAB + 1))))
        dp4 = dp4.reshape(nf, EMB, VPAD // 128, 128)
        prev = dp4
        xdg = xdt[f0:f0 + nf]
        k = _make_group_kernel(nf, i == 0)
        if i == 0:
            out3, wpart = k(xdg, xw, dp4, wt)
        else:
            out3 = k(xdg, dp4)
        pieces.append(out3.transpose(2, 0, 1).reshape(BATCH, nf * EMB))
        f0 += nf
    x_deep = jnp.concatenate(pieces, axis=1)
    out = _finish(wpart)[0, 0]
    return (x_deep, out)


# speedup vs baseline: 1.0794x; 1.0794x over previous
import functools

import jax
import jax.numpy as jnp
from jax import lax
from jax.experimental import pallas as pl
from jax.experimental.pallas import tpu as pltpu
from jax.experimental.pallas import tpu_sc as plsc

N_FIELDS = 26
VOCAB = 100000
EMB = 32
BATCH = 4096
HIST = 20
WIDE_DIM = N_FIELDS * VOCAB

NC = 2
NS = 16
L = 16
NW = NC * NS

VPAD = 782 * 128
BC = 128
NCHUNK = BATCH // BC
FIELD_GROUPS = (26,)

WROWS = 20320
WIDE_PER_W = BATCH * N_FIELDS // NW
WBURST = 32
NWBURST = WIDE_PER_W // WBURST

_mesh = plsc.VectorSubcoreMesh(core_axis_name="c", subcore_axis_name="s")


def _deep_body(nf, xd_hbm, deep_hbm, out_hbm,
               vline_v, idx0_v, idx1_v, outc_v, sem_i, sem_o, wid):
    idx_slots = (idx0_v, idx1_v)

    @pl.loop(0, nf)
    def _task(t):
        gt = wid * nf + t
        f = gt >> 5
        e = gt & 31
        pltpu.sync_copy(deep_hbm.at[f, e], vline_v)
        pltpu.async_copy(xd_hbm.at[f, :, pl.ds(0, BC)], idx0_v, sem_i)
        @pl.when(t > 0)
        def _():
            pltpu.make_async_copy(outc_v, out_hbm.at[f, e], sem_o).wait()

        @pl.loop(0, NCHUNK // 2)
        def _chunkpair(p):
            for s in range(2):
                c = p * 2 + s
                pltpu.make_async_copy(
                    xd_hbm.at[f, :, pl.ds(0, BC)], idx_slots[s], sem_i,
                ).wait()

                @pl.when(c + 1 < NCHUNK)
                def _():
                    pltpu.async_copy(
                        xd_hbm.at[f, :, pl.ds((c + 1) * BC, BC)],
                        idx_slots[1 - s], sem_i)

                idx_v = idx_slots[s]
                for g in range(BC // L):
                    a0 = jnp.zeros((L,), jnp.float32)
                    a1 = jnp.zeros((L,), jnp.float32)
                    for h in range(HIST):
                        iv = idx_v[h, pl.ds(g * L, L)]
                        val = plsc.load_gather(vline_v, [iv >> 7, iv & 127])
                        if h & 1:
                            a1 = a1 + val
                        else:
                            a0 = a0 + val
                    outc_v[pl.ds(c * BC + g * L, L)] = a0 + a1

        pltpu.async_copy(outc_v, out_hbm.at[f, e], sem_o)

    pltpu.make_async_copy(outc_v, out_hbm.at[0, 0], sem_o).wait()


def _wide_body(xw_hbm, wide_hbm, wpart_hbm,
               widx_v, wrow_v, wv0_v, wv1_v, wacc_v, sem_w, wid):
    pltpu.sync_copy(xw_hbm.at[pl.ds(wid * (NWBURST // 4), NWBURST // 4)],
                    widx_v)

    @pl.loop(0, NWBURST // 4)
    def _rowids(r):
        for g in range(128 // L):
            wrow_v[r, pl.ds(g * L, L)] = widx_v[r, pl.ds(g * L, L)] >> 7

    lane = lax.iota(jnp.int32, L)
    wacc_v[...] = jnp.zeros((L,), jnp.float32)
    wv_slots = (wv0_v, wv1_v)
    pltpu.async_copy(wide_hbm.at[wrow_v.at[0, pl.ds(0, WBURST)]], wv0_v,
                     sem_w)

    @pl.loop(0, NWBURST // 2)
    def _wpair(p):
        for s in range(2):
            b = p * 2 + s
            r = b >> 2
            q = (b & 3) * WBURST
            pltpu.make_async_copy(
                wide_hbm.at[wrow_v.at[0, pl.ds(0, WBURST)]], wv_slots[s],
                sem_w).wait()

            @pl.when(b + 1 < NWBURST)
            def _():
                b1 = b + 1
                pltpu.async_copy(
                    wide_hbm.at[wrow_v.at[b1 >> 2,
                                          pl.ds((b1 & 3) * WBURST, WBURST)]],
                    wv_slots[1 - s], sem_w)

            acc = wacc_v[...]
            for g in range(WBURST // L):
                cols = widx_v[r, pl.ds(q + g * L, L)] & 127
                acc = acc + plsc.load_gather(wv_slots[s],
                                             [lane + g * L, cols])
            wacc_v[...] = acc

    pltpu.sync_copy(wacc_v, wpart_hbm.at[wid])


_DEEP_SCRATCH = [
    pltpu.VMEM((VPAD // 128, 128), jnp.float32),
    pltpu.VMEM((HIST, BC), jnp.int32),
    pltpu.VMEM((HIST, BC), jnp.int32),
    pltpu.VMEM((BATCH,), jnp.float32),
    pltpu.SemaphoreType.DMA,
    pltpu.SemaphoreType.DMA,
]
_WIDE_SCRATCH = [
    pltpu.VMEM((NWBURST // 4, 128), jnp.int32),
    pltpu.VMEM((NWBURST // 4, 128), jnp.int32),
    pltpu.VMEM((WBURST, 128), jnp.float32),
    pltpu.VMEM((WBURST, 128), jnp.float32),
    pltpu.VMEM((L,), jnp.float32),
    pltpu.SemaphoreType.DMA,
]
_CP = pltpu.CompilerParams(use_tc_tiling_on_sc=False,
                           needs_layout_passes=False)


@functools.cache
def _make_group_kernel(nf, with_wide):
    if with_wide:
        out_type = (jax.ShapeDtypeStruct((nf, EMB, BATCH), jnp.float32),
                    jax.ShapeDtypeStruct((NW, L), jnp.float32))
        scratch = _DEEP_SCRATCH + _WIDE_SCRATCH

        def body(xd_hbm, xw_hbm, deep_hbm, wide_hbm, out_hbm, wpart_hbm,
                 vline_v, idx0_v, idx1_v, outc_v, sem_i, sem_o,
                 widx_v, wrow_v, wv0_v, wv1_v, wacc_v, sem_w):
            wid = lax.axis_index("s") * NC + lax.axis_index("c")
            _deep_body(nf, xd_hbm, deep_hbm, out_hbm,
                       vline_v, idx0_v, idx1_v, outc_v, sem_i, sem_o, wid)
            _wide_body(xw_hbm, wide_hbm, wpart_hbm,
                       widx_v, wrow_v, wv0_v, wv1_v, wacc_v, sem_w, wid)
    else:
        out_type = jax.ShapeDtypeStruct((nf, EMB, BATCH), jnp.float32)
        scratch = _DEEP_SCRATCH

        def body(xd_hbm, deep_hbm, out_hbm,
                 vline_v, idx0_v, idx1_v, outc_v, sem_i, sem_o):
            wid = lax.axis_index("s") * NC + lax.axis_index("c")
            _deep_body(nf, xd_hbm, deep_hbm, out_hbm,
                       vline_v, idx0_v, idx1_v, outc_v, sem_i, sem_o, wid)

    return pl.kernel(body, out_type=out_type, mesh=_mesh,
                     compiler_params=_CP, scratch_types=scratch)


def _finish_body(p_ref, o_ref):
    o_ref[0, 0] = jax.nn.sigmoid(jnp.sum(p_ref[...]))


_finish = pl.pallas_call(
    _finish_body,
    out_specs=pl.BlockSpec(memory_space=pltpu.SMEM),
    out_shape=jax.ShapeDtypeStruct((1, 1), jnp.float32),
)


def kernel(X_w, X_d, deep_tables, wide_table):
    xdt = X_d.transpose(0, 2, 1)
    xw = X_w.reshape(BATCH * N_FIELDS // 128, 128)
    wt = jnp.pad(wide_table, ((0, WROWS * 128 - (WIDE_DIM + 1)), (0, 0)))
    wt = wt.reshape(WROWS, 128)

    dpt = deep_tables.transpose(0, 2, 1)
    pieces, wpart = [], None
    f0, prev = 0, None
    for i, nf in enumerate(FIELD_GROUPS):
        tbl = dpt
        if prev is not None:
            tbl = lax.optimization_barrier((tbl, prev))[0]
        dp4 = jnp.pad(tbl[f0:f0 + nf],
                      ((0, 0), (0, 0), (0, VPAD - (VOCAB + 1))))
        dp4 = dp4.reshape(nf, EMB, VPAD // 128, 128)
        prev = dp4
        xdg = xdt[f0:f0 + nf]
        k = _make_group_kernel(nf, i == 0)
        if i == 0:
            out3, wpart = k(xdg, xw, dp4, wt)
        else:
            out3 = k(xdg, dp4)
        pieces.append(out3.transpose(2, 0, 1).reshape(BATCH, nf * EMB))
        f0 += nf
    x_deep = jnp.concatenate(pieces, axis=1)
    out = _finish(wpart)[0, 0]
    return (x_deep, out)


# speedup vs baseline: 1.2825x; 1.1882x over previous
import functools

import jax
import jax.numpy as jnp
from jax import lax
from jax.experimental import pallas as pl
from jax.experimental.pallas import tpu as pltpu
from jax.experimental.pallas import tpu_sc as plsc

N_FIELDS = 26
VOCAB = 100000
EMB = 32
BATCH = 4096
HIST = 20
WIDE_DIM = N_FIELDS * VOCAB

NC = 2
NS = 16
L = 16
NW = NC * NS

VPAD = 782 * 128
BC = 128
NCHUNK = BATCH // BC
FIELD_GROUPS = (26,)

WROWS = 20320
WIDE_PER_W = BATCH * N_FIELDS // NW
WBURST = 32
NWBURST = WIDE_PER_W // WBURST

_mesh = plsc.VectorSubcoreMesh(core_axis_name="c", subcore_axis_name="s")


def _deep_body(nf, xd_hbm, deep_hbm, out_hbm,
               vline_v, idx0_v, idx1_v, outc_v, sem_i, sem_o, wid):
    idx_slots = (idx0_v, idx1_v)

    @pl.loop(0, nf)
    def _task(t):
        gt = wid * nf + t
        f = gt >> 5
        e = gt & 31
        pltpu.sync_copy(deep_hbm.at[f, e >> 3, :, e & 7], vline_v)
        pltpu.async_copy(xd_hbm.at[f, :, pl.ds(0, BC)], idx0_v, sem_i)
        @pl.when(t > 0)
        def _():
            pltpu.make_async_copy(outc_v, out_hbm.at[f, e], sem_o).wait()

        @pl.loop(0, NCHUNK // 2)
        def _chunkpair(p):
            for s in range(2):
                c = p * 2 + s
                pltpu.make_async_copy(
                    xd_hbm.at[f, :, pl.ds(0, BC)], idx_slots[s], sem_i,
                ).wait()

                @pl.when(c + 1 < NCHUNK)
                def _():
                    pltpu.async_copy(
                        xd_hbm.at[f, :, pl.ds((c + 1) * BC, BC)],
                        idx_slots[1 - s], sem_i)

                idx_v = idx_slots[s]
                for g in range(BC // L):
                    a0 = jnp.zeros((L,), jnp.float32)
                    a1 = jnp.zeros((L,), jnp.float32)
                    for h in range(HIST):
                        iv = idx_v[h, pl.ds(g * L, L)]
                        val = plsc.load_gather(vline_v, [iv >> 7, iv & 127])
                        if h & 1:
                            a1 = a1 + val
                        else:
                            a0 = a0 + val
                    outc_v[pl.ds(c * BC + g * L, L)] = a0 + a1

        pltpu.async_copy(outc_v, out_hbm.at[f, e], sem_o)

    pltpu.make_async_copy(outc_v, out_hbm.at[0, 0], sem_o).wait()


def _wide_body(xw_hbm, wide_hbm, wpart_hbm,
               widx_v, wrow_v, wv0_v, wv1_v, wacc_v, sem_w, wid):
    pltpu.sync_copy(xw_hbm.at[pl.ds(wid * (NWBURST // 4), NWBURST // 4)],
                    widx_v)

    @pl.loop(0, NWBURST // 4)
    def _rowids(r):
        for g in range(128 // L):
            wrow_v[r, pl.ds(g * L, L)] = widx_v[r, pl.ds(g * L, L)] >> 7

    lane = lax.iota(jnp.int32, L)
    wacc_v[...] = jnp.zeros((L,), jnp.float32)
    wv_slots = (wv0_v, wv1_v)
    pltpu.async_copy(wide_hbm.at[wrow_v.at[0, pl.ds(0, WBURST)]], wv0_v,
                     sem_w)

    @pl.loop(0, NWBURST // 2)
    def _wpair(p):
        for s in range(2):
            b = p * 2 + s
            r = b >> 2
            q = (b & 3) * WBURST
            pltpu.make_async_copy(
                wide_hbm.at[wrow_v.at[0, pl.ds(0, WBURST)]], wv_slots[s],
                sem_w).wait()

            @pl.when(b + 1 < NWBURST)
            def _():
                b1 = b + 1
                pltpu.async_copy(
                    wide_hbm.at[wrow_v.at[b1 >> 2,
                                          pl.ds((b1 & 3) * WBURST, WBURST)]],
                    wv_slots[1 - s], sem_w)

            acc = wacc_v[...]
            for g in range(WBURST // L):
                cols = widx_v[r, pl.ds(q + g * L, L)] & 127
                acc = acc + plsc.load_gather(wv_slots[s],
                                             [lane + g * L, cols])
            wacc_v[...] = acc

    pltpu.sync_copy(wacc_v, wpart_hbm.at[wid])


_DEEP_SCRATCH = [
    pltpu.VMEM((VPAD // 128, 128), jnp.float32),
    pltpu.VMEM((HIST, BC), jnp.int32),
    pltpu.VMEM((HIST, BC), jnp.int32),
    pltpu.VMEM((BATCH,), jnp.float32),
    pltpu.SemaphoreType.DMA,
    pltpu.SemaphoreType.DMA,
]
_WIDE_SCRATCH = [
    pltpu.VMEM((NWBURST // 4, 128), jnp.int32),
    pltpu.VMEM((NWBURST // 4, 128), jnp.int32),
    pltpu.VMEM((WBURST, 128), jnp.float32),
    pltpu.VMEM((WBURST, 128), jnp.float32),
    pltpu.VMEM((L,), jnp.float32),
    pltpu.SemaphoreType.DMA,
]
_CP = pltpu.CompilerParams(use_tc_tiling_on_sc=False,
                           needs_layout_passes=False)


@functools.cache
def _make_group_kernel(nf, with_wide):
    if with_wide:
        out_type = (jax.ShapeDtypeStruct((nf, EMB, BATCH), jnp.float32),
                    jax.ShapeDtypeStruct((NW, L), jnp.float32))
        scratch = _DEEP_SCRATCH + _WIDE_SCRATCH

        def body(xd_hbm, xw_hbm, deep_hbm, wide_hbm, out_hbm, wpart_hbm,
                 vline_v, idx0_v, idx1_v, outc_v, sem_i, sem_o,
                 widx_v, wrow_v, wv0_v, wv1_v, wacc_v, sem_w):
            wid = lax.axis_index("s") * NC + lax.axis_index("c")
            _deep_body(nf, xd_hbm, deep_hbm, out_hbm,
                       vline_v, idx0_v, idx1_v, outc_v, sem_i, sem_o, wid)
            _wide_body(xw_hbm, wide_hbm, wpart_hbm,
                       widx_v, wrow_v, wv0_v, wv1_v, wacc_v, sem_w, wid)
    else:
        out_type = jax.ShapeDtypeStruct((nf, EMB, BATCH), jnp.float32)
        scratch = _DEEP_SCRATCH

        def body(xd_hbm, deep_hbm, out_hbm,
                 vline_v, idx0_v, idx1_v, outc_v, sem_i, sem_o):
            wid = lax.axis_index("s") * NC + lax.axis_index("c")
            _deep_body(nf, xd_hbm, deep_hbm, out_hbm,
                       vline_v, idx0_v, idx1_v, outc_v, sem_i, sem_o, wid)

    return pl.kernel(body, out_type=out_type, mesh=_mesh,
                     compiler_params=_CP, scratch_types=scratch)


def _finish_body(p_ref, o_ref):
    o_ref[0, 0] = jax.nn.sigmoid(jnp.sum(p_ref[...]))


_finish = pl.pallas_call(
    _finish_body,
    out_specs=pl.BlockSpec(memory_space=pltpu.SMEM),
    out_shape=jax.ShapeDtypeStruct((1, 1), jnp.float32),
)


def kernel(X_w, X_d, deep_tables, wide_table):
    xdt = X_d.transpose(0, 2, 1)
    xw = X_w.reshape(BATCH * N_FIELDS // 128, 128)
    wt = jnp.pad(wide_table, ((0, WROWS * 128 - (WIDE_DIM + 1)), (0, 0)))
    wt = wt.reshape(WROWS, 128)

    dpt = deep_tables.transpose(0, 2, 1)
    pieces, wpart = [], None
    f0, prev = 0, None
    for i, nf in enumerate(FIELD_GROUPS):
        tbl = dpt
        if prev is not None:
            tbl = lax.optimization_barrier((tbl, prev))[0]
        dp4 = jnp.pad(tbl[f0:f0 + nf],
                      ((0, 0), (0, 0), (0, VPAD - (VOCAB + 1))))
        dp4 = dp4.reshape(nf, 4, 8, VPAD // 128, 128).transpose(0, 1, 3, 2, 4)
        prev = dp4
        xdg = xdt[f0:f0 + nf]
        k = _make_group_kernel(nf, i == 0)
        if i == 0:
            out3, wpart = k(xdg, xw, dp4, wt)
        else:
            out3 = k(xdg, dp4)
        pieces.append(out3.transpose(2, 0, 1).reshape(BATCH, nf * EMB))
        f0 += nf
    x_deep = jnp.concatenate(pieces, axis=1)
    out = _finish(wpart)[0, 0]
    return (x_deep, out)
